# baseline (speedup 1.0000x reference)
import functools

import jax
import jax.numpy as jnp
from jax import lax
from jax.experimental import pallas as pl
from jax.experimental.pallas import tpu as pltpu
from jax.experimental.pallas import tpu_sc as plsc


def _dist_kernel(a_ref, b_ref, o_ref, *, L):
    gblk = a_ref.shape[1]
    shape = (gblk, 128)
    one = jnp.int32(1)
    a = [a_ref[j] for j in range(L)]

    Pv = jnp.full(shape, (1 << L) - 1, jnp.int32)
    Mv = jnp.zeros(shape, jnp.int32)
    score = jnp.full(shape, L, jnp.int32)
    for i in range(L):
        bi = b_ref[i]
        Eq = jnp.zeros(shape, jnp.int32)
        for j in range(L):
            Eq = Eq | jnp.where(a[j] == bi, jnp.int32(1 << j), jnp.int32(0))
        Xv = Eq | Mv
        Xh = (((Eq & Pv) + Pv) ^ Pv) | Eq
        Ph = Mv | ~(Xh | Pv)
        Mh = Pv & Xh
        score = score + ((Ph >> (L - 1)) & one) - ((Mh >> (L - 1)) & one)
        Ph = (Ph << 1) | one
        Mh = Mh << 1
        Pv = Mh | ~(Xv | Ph)
        Mv = Ph & Xv
    o_ref[...] = score


def _make_sc_lookup(B, D, b_per_w, num_cores):
    mesh = plsc.VectorSubcoreMesh(core_axis_name="c", subcore_axis_name="s")
    n_vec = b_per_w * D // 16

    @functools.partial(
        pl.kernel,
        mesh=mesh,
        out_type=jax.ShapeDtypeStruct((B * D,), jnp.float32),
        compiler_params=pltpu.CompilerParams(needs_layout_passes=False),
        scratch_types=[
            pltpu.VMEM((b_per_w,), jnp.int32),
            pltpu.VMEM((128,), jnp.float32),
            pltpu.VMEM((b_per_w * D,), jnp.float32),
        ],
    )
    def sc_lookup(table_hbm, idx_hbm, out_hbm, idx_v, head_v, out_v):
        wid = lax.axis_index("s") * num_cores + lax.axis_index("c")
        base = wid * b_per_w
        pltpu.sync_copy(table_hbm.at[pl.ds(0, 128)], head_v)
        pltpu.sync_copy(idx_hbm.at[pl.ds(base, b_per_w)], idx_v)
        ii = lax.iota(jnp.int32, 16)
        q = ii >> 2
        r = ii & 3
        for i in range(n_vec):
            ids = plsc.load_gather(idx_v, [q + (i * 4)])
            vals = plsc.load_gather(head_v, [ids * 4 + r])
            out_v[pl.ds(i * 16, 16)] = vals
        pltpu.sync_copy(out_v, out_hbm.at[pl.ds(base * D, b_per_w * D)])

    return sc_lookup


def kernel(input1, input2, embedding_table):
    B, L = input1.shape
    G = B // 128
    grid = 8
    gblk = G // grid
    a3 = input1.T.reshape(L, G, 128)
    b3 = input2.T.reshape(L, G, 128)
    dist = pl.pallas_call(
        functools.partial(_dist_kernel, L=L),
        grid=(grid,),
        in_specs=[
            pl.BlockSpec((L, gblk, 128), lambda g: (0, g, 0)),
            pl.BlockSpec((L, gblk, 128), lambda g: (0, g, 0)),
        ],
        out_specs=pl.BlockSpec((gblk, 128), lambda g: (g, 0)),
        out_shape=jax.ShapeDtypeStruct((G, 128), jnp.int32),
    )(a3, b3)
    ids = dist.reshape(B)

    info = plsc.get_sparse_core_info()
    nw = info.num_cores * info.num_subcores
    b_per_w = B // nw
    D = embedding_table.shape[1]
    table_flat = embedding_table.reshape(-1)
    sc_lookup = _make_sc_lookup(B, D, b_per_w, info.num_cores)
    out = sc_lookup(table_flat, ids)
    return out.reshape(B, D)


# speedup vs baseline: 3.3888x; 3.3888x over previous
import functools

import jax
import jax.numpy as jnp
from jax.experimental import pallas as pl


def _edit_kernel(ab_ref, t_ref, o_ref, *, L):
    gblk = ab_ref.shape[2]
    shape = (gblk, 128)
    one = jnp.int32(1)
    a = [ab_ref[0, j] for j in range(L)]

    Pv = jnp.full(shape, (1 << L) - 1, jnp.int32)
    Mv = jnp.zeros(shape, jnp.int32)
    score = jnp.full(shape, L, jnp.int32)
    for i in range(L):
        bi = ab_ref[1, i]
        Eq = jnp.zeros(shape, jnp.int32)
        for j in range(L):
            Eq = Eq | jnp.where(a[j] == bi, jnp.int32(1 << j), jnp.int32(0))
        Xv = Eq | Mv
        Xh = (((Eq & Pv) + Pv) ^ Pv) | Eq
        Ph = Mv | ~(Xh | Pv)
        Mh = Pv & Xh
        score = score + ((Ph >> (L - 1)) & one) - ((Mh >> (L - 1)) & one)
        Ph = (Ph << 1) | one
        Mh = Mh << 1
        Pv = Mh | ~(Xv | Ph)
        Mv = Ph & Xv

    for d in range(4):
        acc = jnp.zeros(shape, jnp.float32)
        for k in range(L + 1):
            acc = jnp.where(score == k, t_ref[k, d], acc)
        o_ref[d] = acc


def kernel(input1, input2, embedding_table):
    B, L = input1.shape
    G = B // 128
    grid = 8
    gblk = G // grid
    ab = jnp.concatenate([input1, input2], axis=1).T.reshape(2, L, G, 128)
    out = pl.pallas_call(
        functools.partial(_edit_kernel, L=L),
        grid=(grid,),
        in_specs=[
            pl.BlockSpec((2, L, gblk, 128), lambda g: (0, 0, g, 0)),
            pl.BlockSpec((32, 4), lambda g: (0, 0)),
        ],
        out_specs=pl.BlockSpec((4, gblk, 128), lambda g: (0, g, 0)),
        out_shape=jax.ShapeDtypeStruct((4, G, 128), jnp.float32),
    )(ab, embedding_table)
    return out.transpose(1, 2, 0).reshape(B, 4)


# speedup vs baseline: 3.9224x; 1.1575x over previous
import functools

import jax
import jax.numpy as jnp
from jax.experimental import pallas as pl


def _edit_kernel(ab_ref, t_ref, o_ref, *, L):
    gblk = ab_ref.shape[2]
    shape = (gblk, 128)
    one = jnp.int32(1)
    a = [ab_ref[0, j].astype(jnp.int32) for j in range(L)]

    Pv = jnp.full(shape, (1 << L) - 1, jnp.int32)
    Mv = jnp.zeros(shape, jnp.int32)
    score = jnp.full(shape, L, jnp.int32)
    for i in range(L):
        bi = ab_ref[1, i].astype(jnp.int32)
        Eq = jnp.zeros(shape, jnp.int32)
        for j in range(L):
            Eq = Eq | jnp.where(a[j] == bi, jnp.int32(1 << j), jnp.int32(0))
        Xv = Eq | Mv
        Xh = (((Eq & Pv) + Pv) ^ Pv) | Eq
        Ph = Mv | ~(Xh | Pv)
        Mh = Pv & Xh
        score = score + ((Ph >> (L - 1)) & one) - ((Mh >> (L - 1)) & one)
        Ph = (Ph << 1) | one
        Mh = Mh << 1
        Pv = Mh | ~(Xv | Ph)
        Mv = Ph & Xv

    for d in range(4):
        acc = jnp.zeros(shape, jnp.float32)
        for k in range(L + 1):
            acc = jnp.where(score == k, t_ref[k, d], acc)
        o_ref[d] = acc


def kernel(input1, input2, embedding_table):
    B, L = input1.shape
    G = B // 128
    grid = 8
    gblk = G // grid
    ab = jnp.concatenate([input1, input2], axis=1).T.reshape(2, L, G, 128)
    out = pl.pallas_call(
        functools.partial(_edit_kernel, L=L),
        grid=(grid,),
        in_specs=[
            pl.BlockSpec((2, L, gblk, 128), lambda g: (0, 0, g, 0)),
            pl.BlockSpec((32, 4), lambda g: (0, 0)),
        ],
        out_specs=pl.BlockSpec((4, gblk, 128), lambda g: (0, g, 0)),
        out_shape=jax.ShapeDtypeStruct((4, G, 128), jnp.float32),
    )(ab, embedding_table)
    return out.transpose(1, 2, 0).reshape(B, 4)
